# Initial kernel scaffold; baseline (speedup 1.0000x reference)
import jax
import jax.numpy as jnp
from jax.experimental import pallas as pl


def kernel(batch_x, ts, x, last_update, edge_index, batch_vec, time_w, time_b, tag_w0, tag_w1, tag_b, rg_wk, rg_wq, rg_wv, rg_ws, rg_b, lp_w1, lp_b1, lp_w2, lp_b2):
    raise NotImplementedError("write your pallas kernel here")


# speedup vs baseline: 7.4514x; 7.4514x over previous
import functools

import jax
import jax.numpy as jnp
from jax import lax
from jax.experimental import pallas as pl
from jax.experimental.pallas import tpu as pltpu
from jax.experimental.pallas import tpu_sc as plsc

N = 10000
E = 320000
G = 512
NP = 10240
BLK = 1280
NBLK = NP // BLK
NBE = 80
NBE2 = 160
EPW = NBE * 128
EP = 32 * EPW
NBB = 3
BP = 32 * NBB * 128
CNT_PAD = 640
ROWS_PER_TILE = NP // 16

_f32 = jnp.float32
_i32 = jnp.int32


def _mmw(a, w):
    return jnp.dot(a.astype(jnp.bfloat16), w.astype(jnp.bfloat16),
                   preferred_element_type=_f32)


def _tc1_body(bx, ts, x, tw, tb, w0, deg, g_o, o0_o):
    i = pl.program_id(0)
    rows = lax.broadcasted_iota(_i32, (BLK, 1), 0) + i * BLK
    m = (rows < N).astype(_f32)
    rt = jnp.cos(ts[...] * tw[...] + tb[...])
    r64 = lax.broadcasted_iota(_i32, (64, 128), 0)
    c64 = lax.broadcasted_iota(_i32, (64, 128), 1)
    p1 = (c64 == r64).astype(_f32)
    r32 = lax.broadcasted_iota(_i32, (32, 128), 0)
    c32 = lax.broadcasted_iota(_i32, (32, 128), 1)
    p2 = (c32 == r32 + 64).astype(_f32)
    p3 = (c32 == r32 + 96).astype(_f32)
    h = (jnp.dot(bx[...], p1, preferred_element_type=_f32, precision=lax.Precision.HIGHEST)
         + jnp.dot(rt, p2, preferred_element_type=_f32, precision=lax.Precision.HIGHEST)
         + jnp.dot(x[...], p3, preferred_element_type=_f32, precision=lax.Precision.HIGHEST))
    h = h * m
    d = deg[...]
    dis = jnp.where(d > 0, 1.0 / jnp.sqrt(jnp.maximum(d, 1e-12)), 0.0)
    g_o[...] = dis * h
    o0_o[...] = _mmw(h, w0[...])


def _tc1(bx, ts, x, tw, tb, w0, deg):
    return pl.pallas_call(
        _tc1_body,
        grid=(NBLK,),
        in_specs=[
            pl.BlockSpec((BLK, 64), lambda i: (i, 0)),
            pl.BlockSpec((BLK, 1), lambda i: (i, 0)),
            pl.BlockSpec((BLK, 32), lambda i: (i, 0)),
            pl.BlockSpec((1, 32), lambda i: (0, 0)),
            pl.BlockSpec((1, 32), lambda i: (0, 0)),
            pl.BlockSpec((128, 128), lambda i: (0, 0)),
            pl.BlockSpec((BLK, 1), lambda i: (i, 0)),
        ],
        out_specs=[pl.BlockSpec((BLK, 128), lambda i: (i, 0))] * 2,
        out_shape=[jax.ShapeDtypeStruct((NP, 128), _f32)] * 2,
    )(bx, ts, x, tw, tb, w0, deg)


def _tc2_body(s0, s1, deg, o0, w1, tb, wk, wq, wv, h1_o, k_o, q_o, v_o):
    i = pl.program_id(0)
    rows = lax.broadcasted_iota(_i32, (BLK, 1), 0) + i * BLK
    m = (rows < N).astype(_f32)
    d = deg[...]
    dis = jnp.where(d > 0, 1.0 / jnp.sqrt(jnp.maximum(d, 1e-12)), 0.0)
    sp = jnp.reshape(s0[...], (BLK, 128)) + jnp.reshape(s1[...], (BLK, 128))
    hp = dis * sp
    h1 = o0[...] + _mmw(hp, w1[...]) + tb[...]
    h1 = h1 * m
    h1_o[...] = h1
    k_o[...] = _mmw(h1, wk[...])
    q_o[...] = _mmw(h1, wq[...])
    v_o[...] = _mmw(h1, wv[...])


def _tc2(s_parts, deg, o0, w1, tb, wk, wq, wv):
    return pl.pallas_call(
        _tc2_body,
        grid=(NBLK,),
        in_specs=[
            pl.BlockSpec((1, BLK, 128), lambda i: (0, i, 0)),
            pl.BlockSpec((1, BLK, 128), lambda i: (1, i, 0)),
            pl.BlockSpec((BLK, 1), lambda i: (i, 0)),
            pl.BlockSpec((BLK, 128), lambda i: (i, 0)),
            pl.BlockSpec((128, 128), lambda i: (0, 0)),
            pl.BlockSpec((1, 128), lambda i: (0, 0)),
            pl.BlockSpec((128, 128), lambda i: (0, 0)),
            pl.BlockSpec((128, 128), lambda i: (0, 0)),
            pl.BlockSpec((128, 128), lambda i: (0, 0)),
        ],
        out_specs=[pl.BlockSpec((BLK, 128), lambda i: (i, 0))] * 4,
        out_shape=[jax.ShapeDtypeStruct((NP, 128), _f32)] * 4,
    )(s_parts, s_parts, deg, o0, w1, tb, wk, wq, wv)


def _tc3_body(cnt, a0, a1, h1b, ws, rgb, w1, b1, w2, b2, out, xa, xh):
    i = pl.program_id(0)

    @pl.when(i == 0)
    def _():
        xa[...] = jnp.zeros((G, 128), _f32)
        xh[...] = jnp.zeros((G, 128), _f32)

    rg = lax.broadcasted_iota(_i32, (G, G), 0)
    cg = lax.broadcasted_iota(_i32, (G, G), 1)
    tri = (cg < rg).astype(_f32)
    roots_f = jnp.dot(tri, cnt[...], preferred_element_type=_f32, precision=lax.Precision.HIGHEST)
    roots = jnp.minimum(roots_f.astype(_i32), N - 1)
    cols = lax.broadcasted_iota(_i32, (G, BLK), 1) + i * BLK
    oh = (cols == roots).astype(_f32)
    ab = jnp.reshape(a0[...], (BLK, 128)) + jnp.reshape(a1[...], (BLK, 128))
    xa[...] += jnp.dot(oh, ab, preferred_element_type=_f32, precision=lax.Precision.HIGHEST)
    xh[...] += jnp.dot(oh, h1b[...], preferred_element_type=_f32, precision=lax.Precision.HIGHEST)

    @pl.when(i == NBLK - 1)
    def _():
        h2r = xa[...] + _mmw(xh[...], ws[...]) + rgb[...]
        hid = jnp.maximum(_mmw(h2r, w1[...]) + b1[...], 0.0)
        out[...] = _mmw(hid, w2[...]) + b2[...]


def _tc3(cnt, agg_parts, h1, ws, rgb, w1, b1, w2, b2):
    return pl.pallas_call(
        _tc3_body,
        grid=(NBLK,),
        in_specs=[
            pl.BlockSpec((G, 1), lambda i: (0, 0)),
            pl.BlockSpec((1, BLK, 128), lambda i: (0, i, 0)),
            pl.BlockSpec((1, BLK, 128), lambda i: (1, i, 0)),
            pl.BlockSpec((BLK, 128), lambda i: (i, 0)),
            pl.BlockSpec((128, 128), lambda i: (0, 0)),
            pl.BlockSpec((1, 128), lambda i: (0, 0)),
            pl.BlockSpec((128, 128), lambda i: (0, 0)),
            pl.BlockSpec((1, 128), lambda i: (0, 0)),
            pl.BlockSpec((128, 1), lambda i: (0, 0)),
            pl.BlockSpec((1, 1), lambda i: (0, 0)),
        ],
        out_specs=pl.BlockSpec((G, 1), lambda i: (0, 0)),
        out_shape=jax.ShapeDtypeStruct((G, 1), _f32),
        scratch_shapes=[pltpu.VMEM((G, 128), _f32), pltpu.VMEM((G, 128), _f32)],
    )(cnt, agg_parts, agg_parts, h1, ws, rgb, w1, b1, w2, b2)


_MESH = plsc.VectorSubcoreMesh(core_axis_name="c", subcore_axis_name="s")


def _zero_vmem_2d(buf, rows):
    def zb(i, _):
        for t in range(8):
            buf[i, pl.ds(t * 16, 16)] = jnp.zeros((16,), _f32)
        return 0
    lax.fori_loop(0, rows, zb, 0)


def _sc_degcnt_body(dst_w, bv_w, deg_out, cnt_out, deg_sp, cnt_sp, idx_v,
                    bidx_v, ones_v, zbuf):
    c = lax.axis_index("c")
    s = lax.axis_index("s")
    for t in range(ROWS_PER_TILE // 16):
        zbuf[pl.ds(t * 16, 16)] = jnp.zeros((16,), _f32)
    for t in range(8):
        ones_v[pl.ds(t * 16, 16)] = jnp.full((16,), 1.0, _f32)
    pltpu.sync_copy(zbuf, deg_sp.at[pl.ds(s * ROWS_PER_TILE, ROWS_PER_TILE)])

    @pl.when(s == 0)
    def _():
        pltpu.sync_copy(zbuf.at[pl.ds(0, CNT_PAD)], cnt_sp)

    pltpu.sync_copy(dst_w.at[c, s], idx_v)
    pltpu.sync_copy(bv_w.at[c, s], bidx_v)
    plsc.subcore_barrier()

    def body(j, carry):
        pltpu.sync_copy(ones_v, deg_sp.at[idx_v.at[j]], add=True)
        return carry
    lax.fori_loop(0, NBE, body, 0)
    for j in range(NBB):
        pltpu.sync_copy(ones_v, cnt_sp.at[bidx_v.at[j]], add=True)
    plsc.subcore_barrier()

    pltpu.sync_copy(deg_sp.at[pl.ds(s * ROWS_PER_TILE, ROWS_PER_TILE)], zbuf)
    pltpu.sync_copy(zbuf, deg_out.at[c, pl.ds(s * ROWS_PER_TILE, ROWS_PER_TILE)])

    @pl.when(s == 0)
    def _():
        pltpu.sync_copy(cnt_sp, zbuf.at[pl.ds(0, CNT_PAD)])
        pltpu.sync_copy(zbuf.at[pl.ds(0, CNT_PAD)], cnt_out.at[c])


@functools.partial(
    pl.kernel,
    out_type=(jax.ShapeDtypeStruct((2, NP), _f32),
              jax.ShapeDtypeStruct((2, CNT_PAD), _f32)),
    mesh=_MESH,
    scratch_types=[
        pltpu.VMEM_SHARED((NP,), _f32),
        pltpu.VMEM_SHARED((CNT_PAD,), _f32),
        pltpu.VMEM((NBE, 128), _i32),
        pltpu.VMEM((NBB, 128), _i32),
        pltpu.VMEM((128,), _f32),
        pltpu.VMEM((ROWS_PER_TILE,), _f32),
    ],
)
def _sparse_degcnt(*args):
    _sc_degcnt_body(*args)


def _sc_round1_body(g_hbm, src_w, dst_w, s_out, acc_sp, sidx, didx, gbuf):
    c = lax.axis_index("c")
    s = lax.axis_index("s")
    _zero_vmem_2d(gbuf, 128)
    for t in range(ROWS_PER_TILE // 128):
        pltpu.sync_copy(gbuf, acc_sp.at[pl.ds(s * ROWS_PER_TILE + t * 128, 128)])
    plsc.subcore_barrier()

    def outer(jo, carry):
        pltpu.sync_copy(src_w.at[c, s, pl.ds(jo * 16, 16)], sidx)
        pltpu.sync_copy(dst_w.at[c, s, pl.ds(jo * 16, 16)], didx)

        def body(j, ic):
            pltpu.sync_copy(g_hbm.at[sidx.at[j]], gbuf)
            pltpu.sync_copy(gbuf, acc_sp.at[didx.at[j]], add=True)
            return ic
        lax.fori_loop(0, 16, body, 0)
        return carry
    lax.fori_loop(0, NBE // 16, outer, 0)
    plsc.subcore_barrier()

    for t in range(ROWS_PER_TILE // 128):
        r0 = s * ROWS_PER_TILE + t * 128
        pltpu.sync_copy(acc_sp.at[pl.ds(r0, 128)], gbuf)
        pltpu.sync_copy(gbuf, s_out.at[c, pl.ds(r0, 128)])


@functools.partial(
    pl.kernel,
    out_type=jax.ShapeDtypeStruct((2, NP, 128), _f32),
    mesh=_MESH,
    scratch_types=[
        pltpu.VMEM_SHARED((NP, 128), _f32),
        pltpu.VMEM((16, 128), _i32),
        pltpu.VMEM((16, 128), _i32),
        pltpu.VMEM((128, 128), _f32),
    ],
)
def _sparse_round1(*args):
    _sc_round1_body(*args)


def _sc_round2_body(k_hbm, q_hbm, v_hbm, src_w, dst_w, agg_out, acc_sp,
                    sidx, didx, kb, qb):
    c = lax.axis_index("c")
    s = lax.axis_index("s")
    _zero_vmem_2d(kb, 64)
    for t in range(ROWS_PER_TILE // 64):
        pltpu.sync_copy(kb, acc_sp.at[pl.ds(s * ROWS_PER_TILE + t * 64, 64)])
    plsc.subcore_barrier()

    def outer(jo, carry):
        pltpu.sync_copy(src_w.at[c, s, pl.ds(jo * 16, 16)], sidx)
        pltpu.sync_copy(dst_w.at[c, s, pl.ds(jo * 16, 16)], didx)

        def body(j, ic):
            pltpu.sync_copy(k_hbm.at[didx.at[j]], kb)
            pltpu.sync_copy(q_hbm.at[sidx.at[j]], qb)

            def rowf1(i, rc):
                for t in range(8):
                    slc = pl.ds(t * 16, 16)
                    xx = kb[i, slc] + qb[i, slc]
                    kb[i, slc] = 1.0 / (1.0 + jnp.exp(-xx))
                return rc
            lax.fori_loop(0, 64, rowf1, 0)
            pltpu.sync_copy(v_hbm.at[sidx.at[j]], qb)

            def rowf2(i, rc):
                for t in range(8):
                    slc = pl.ds(t * 16, 16)
                    kb[i, slc] = kb[i, slc] * qb[i, slc]
                return rc
            lax.fori_loop(0, 64, rowf2, 0)
            pltpu.sync_copy(kb, acc_sp.at[didx.at[j]], add=True)
            return ic
        lax.fori_loop(0, 16, body, 0)
        return carry
    lax.fori_loop(0, NBE2 // 16, outer, 0)
    plsc.subcore_barrier()

    for t in range(ROWS_PER_TILE // 64):
        r0 = s * ROWS_PER_TILE + t * 64
        pltpu.sync_copy(acc_sp.at[pl.ds(r0, 64)], kb)
        pltpu.sync_copy(kb, agg_out.at[c, pl.ds(r0, 64)])


@functools.partial(
    pl.kernel,
    out_type=jax.ShapeDtypeStruct((2, NP, 128), _f32),
    mesh=_MESH,
    scratch_types=[
        pltpu.VMEM_SHARED((NP, 128), _f32),
        pltpu.VMEM((16, 64), _i32),
        pltpu.VMEM((16, 64), _i32),
        pltpu.VMEM((64, 128), _f32),
        pltpu.VMEM((64, 128), _f32),
    ],
)
def _sparse_round2(*args):
    _sc_round2_body(*args)


def kernel(batch_x, ts, x, last_update, edge_index, batch_vec, time_w, time_b,
           tag_w0, tag_w1, tag_b, rg_wk, rg_wq, rg_wv, rg_ws, rg_b,
           lp_w1, lp_b1, lp_w2, lp_b2):
    del last_update
    pe = jnp.arange(EP - E, dtype=_i32)
    src_p = jnp.concatenate([edge_index[0], N + (pe % 240)])
    dst_p = jnp.concatenate([edge_index[1], N + (pe % 240)])
    src_w = src_p.reshape(2, 16, NBE, 128)
    dst_w = dst_p.reshape(2, 16, NBE, 128)
    src_w2 = src_p.reshape(2, 16, NBE2, 64)
    dst_w2 = dst_p.reshape(2, 16, NBE2, 64)
    pb = jnp.arange(BP - N, dtype=_i32)
    bv_w = jnp.concatenate([batch_vec, G + (pb % 16)]).reshape(2, 16, NBB, 128)

    bx_p = jnp.pad(batch_x, ((0, NP - N), (0, 0)))
    ts_p = jnp.pad(ts, (0, NP - N))[:, None]
    x_p = jnp.pad(x, ((0, NP - N), (0, 0)))
    tw = time_w[None, :]
    tbc = time_b[None, :]
    tagb = tag_b[None, :]
    rgb = rg_b[None, :]
    b1 = lp_b1[None, :]
    b2 = lp_b2[None, :]

    deg2, cnt2 = _sparse_degcnt(dst_w, bv_w)
    deg = (deg2[0] + deg2[1])[:, None]
    cnt = (cnt2[0, :G] + cnt2[1, :G])[:, None]

    g, out0 = _tc1(bx_p, ts_p, x_p, tw, tbc, tag_w0, deg)

    s_parts = _sparse_round1(g, src_w, dst_w)

    h1, k, q, v = _tc2(s_parts, deg, out0, tag_w1, tagb, rg_wk, rg_wq, rg_wv)

    agg_parts = _sparse_round2(k, q, v, src_w2, dst_w2)

    return _tc3(cnt, agg_parts, h1, rg_ws, rgb, lp_w1, b1, lp_w2, b2)
